# baseline (speedup 1.0000x reference)
import functools

import jax
import jax.numpy as jnp
from jax import lax
from jax.experimental import pallas as pl
from jax.experimental.pallas import tpu as pltpu
from jax.experimental.pallas import tpu_sc as plsc

_POOL = 64
_LEN = 16
_DIM = 4096
_K = 8
_B = 4
_NW = 32


def _row_ss(a):
    y = a * a
    acc = y[:, 0:128]
    for i in range(1, 32):
        acc = acc + y[:, i * 128:(i + 1) * 128]
    b = acc[:, 0:8]
    for j in range(1, 16):
        b = b + acc[:, j * 8:(j + 1) * 8]
    u = b[:, 0:4] + b[:, 4:8]
    t2 = u[:, 0:2] + u[:, 2:4]
    return t2[:, 0:1] + t2[:, 1:2]


def _select_body(cls_ref, pk_ref, idxrows_ref, sim_ref):
    x = cls_ref[...]
    xn = x * lax.rsqrt(jnp.maximum(_row_ss(x), 1e-12))
    pk = pk_ref[...]
    pkn = pk * lax.rsqrt(jnp.maximum(_row_ss(pk), 1e-12))
    sim = lax.dot_general(
        xn, pkn, (((1,), (1,)), ((), ())),
        preferred_element_type=jnp.float32,
    )
    colid = lax.broadcasted_iota(jnp.int32, (_B, _POOL), 1)
    iota16 = lax.broadcasted_iota(jnp.int32, (_B, _LEN), 1)
    s = sim
    for k in range(_K):
        m = jnp.max(s, axis=1, keepdims=True)
        cand = jnp.where(s >= m, colid, _POOL)
        ik = jnp.min(cand, axis=1, keepdims=True)
        chosen = colid == ik
        s = jnp.where(chosen, -jnp.inf, s)
        idxrows_ref[:, k * _LEN:(k + 1) * _LEN] = ik * _LEN + iota16
        row = lax.dot_general(
            chosen.astype(jnp.float32), pkn, (((1,), (0,)), ((), ())),
            preferred_element_type=jnp.float32, precision=lax.Precision.HIGHEST,
        )
        sim_ref[:, k * _DIM:(k + 1) * _DIM] = row * xn


def _select(cls_features, prompt_key):
    return pl.pallas_call(
        _select_body,
        out_shape=[
            jax.ShapeDtypeStruct((_B, _K * _LEN), jnp.int32),
            jax.ShapeDtypeStruct((_B, _K * _DIM), jnp.float32),
        ],
    )(cls_features, prompt_key)


@functools.cache
def _make_gather_sc():
    @functools.partial(
        pl.kernel,
        mesh=plsc.VectorSubcoreMesh(core_axis_name="c", subcore_axis_name="s"),
        out_type=jax.ShapeDtypeStruct((_B * _K * _LEN, _DIM), jnp.float32),
        scratch_types=[
            pltpu.VMEM((_LEN,), jnp.int32),
            pltpu.VMEM((_LEN, _DIM), jnp.float32),
            pltpu.SemaphoreType.DMA,
        ],
    )
    def _gather_sc(idxrows_hbm, table_hbm, out_hbm, rowidx_v, rows_v, sem):
        wid = lax.axis_index("s") * 2 + lax.axis_index("c")
        pltpu.sync_copy(idxrows_hbm.at[wid], rowidx_v)
        pltpu.async_copy(table_hbm.at[rowidx_v], rows_v, sem).wait()
        pltpu.sync_copy(rows_v, out_hbm.at[pl.ds(wid * _LEN, _LEN)])

    return _gather_sc


def kernel(x_embed, cls_features, prompt, prompt_key):
    del x_embed
    idxrows, sim_flat = _select(cls_features, prompt_key)
    table = prompt.reshape(_POOL * _LEN, _DIM)
    bp = _make_gather_sc()(idxrows.reshape(_NW, _LEN), table)
    return (
        bp.reshape(_B, _K * _LEN, _DIM),
        sim_flat.reshape(_B, _K, _DIM),
    )


# speedup vs baseline: 1.1026x; 1.1026x over previous
import functools

import jax
import jax.numpy as jnp
from jax import lax
from jax.experimental import pallas as pl
from jax.experimental.pallas import tpu as pltpu
from jax.experimental.pallas import tpu_sc as plsc

_POOL = 64
_LEN = 16
_DIM = 4096
_K = 8
_B = 4
_NW = 32
_CH = 4
_NCH = _LEN // _CH


def _row_ss(a):
    y = a * a
    acc = y[:, 0:128]
    for i in range(1, 32):
        acc = acc + y[:, i * 128:(i + 1) * 128]
    b = acc[:, 0:8]
    for j in range(1, 16):
        b = b + acc[:, j * 8:(j + 1) * 8]
    u = b[:, 0:4] + b[:, 4:8]
    t2 = u[:, 0:2] + u[:, 2:4]
    return t2[:, 0:1] + t2[:, 1:2]


def _select_body(cls_ref, pk_ref, idxrows_ref, sim_ref):
    x = cls_ref[...]
    xn = x * lax.rsqrt(jnp.maximum(_row_ss(x), 1e-12))
    pk = pk_ref[...]
    pkn = pk * lax.rsqrt(jnp.maximum(_row_ss(pk), 1e-12))
    sim = lax.dot_general(
        xn, pkn, (((1,), (1,)), ((), ())),
        preferred_element_type=jnp.float32,
    )
    colid = lax.broadcasted_iota(jnp.int32, (_B, _POOL), 1)
    kid = lax.broadcasted_iota(jnp.int32, (_B, _K, _POOL), 1)
    kid16 = lax.broadcasted_iota(jnp.int32, (_B, _K, _NCH, _CH), 1)
    liota = (lax.broadcasted_iota(jnp.int32, (_B, _K, _NCH, _CH), 2) * _CH
             + lax.broadcasted_iota(jnp.int32, (_B, _K, _NCH, _CH), 3))
    oh = jnp.zeros((_B, _K, _POOL), jnp.float32)
    pid = jnp.zeros((_B, _K, _NCH, _CH), jnp.int32)
    s = sim
    for k in range(_K):
        m = jnp.max(s, axis=1, keepdims=True)
        cand = jnp.where(s >= m, colid, _POOL)
        ik = jnp.min(cand, axis=1, keepdims=True)
        chosen = colid == ik
        s = jnp.where(chosen, -jnp.inf, s)
        oh = jnp.where(kid == k, chosen[:, None, :].astype(jnp.float32), oh)
        pid = jnp.where(kid16 == k, ik[:, :, None, None], pid)
    idxrows_ref[...] = pid * _LEN + liota
    rows = lax.dot_general(
        oh.reshape(_B * _K, _POOL), pkn, (((1,), (0,)), ((), ())),
        preferred_element_type=jnp.float32, precision=lax.Precision.HIGHEST,
    )
    sim_ref[...] = rows.reshape(_B, _K, _DIM) * xn[:, None, :]


def _select(cls_features, prompt_key):
    return pl.pallas_call(
        _select_body,
        out_shape=[
            jax.ShapeDtypeStruct((_B, _K, _NCH, _CH), jnp.int32),
            jax.ShapeDtypeStruct((_B, _K, _DIM), jnp.float32),
        ],
    )(cls_features, prompt_key)


@functools.cache
def _make_gather_sc():
    @functools.partial(
        pl.kernel,
        mesh=plsc.VectorSubcoreMesh(core_axis_name="c", subcore_axis_name="s"),
        out_type=jax.ShapeDtypeStruct((_B * _K * _LEN, _DIM), jnp.float32),
        scratch_types=[
            pltpu.VMEM((_NCH, _CH), jnp.int32),
            pltpu.VMEM((_NCH, _CH, _DIM), jnp.float32),
        ] + [pltpu.SemaphoreType.DMA] * (2 * _NCH),
    )
    def _gather_sc(idxrows_hbm, table_hbm, out_hbm, rowidx_v, rows_v, *sems):
        gsems, wsems = sems[:_NCH], sems[_NCH:]
        wid = lax.axis_index("s") * 2 + lax.axis_index("c")
        pltpu.sync_copy(idxrows_hbm.at[wid // _K, wid % _K], rowidx_v)
        gathers = []
        for c in range(_NCH):
            gathers.append(pltpu.async_copy(
                table_hbm.at[rowidx_v.at[c]],
                rows_v.at[c], gsems[c]))
        writes = []
        for c in range(_NCH):
            gathers[c].wait()
            writes.append(pltpu.async_copy(
                rows_v.at[c],
                out_hbm.at[pl.ds(wid * _LEN + c * _CH, _CH)], wsems[c]))
        for wdma in writes:
            wdma.wait()

    return _gather_sc


def kernel(x_embed, cls_features, prompt, prompt_key):
    del x_embed
    idxrows, sim_out = _select(cls_features, prompt_key)
    table = prompt.reshape(_POOL * _LEN, _DIM)
    bp = _make_gather_sc()(idxrows, table)
    return (bp.reshape(_B, _K * _LEN, _DIM), sim_out)


# speedup vs baseline: 1.1313x; 1.0260x over previous
import functools

import jax
import jax.numpy as jnp
from jax import lax
from jax.experimental import pallas as pl
from jax.experimental.pallas import tpu as pltpu
from jax.experimental.pallas import tpu_sc as plsc

_POOL = 64
_LEN = 16
_DIM = 4096
_K = 8
_B = 4
_NW = 32
_CH = 8
_NCH = _LEN // _CH


def _row_ss(a):
    y = a * a
    acc = y[:, 0:128]
    for i in range(1, 32):
        acc = acc + y[:, i * 128:(i + 1) * 128]
    b = acc[:, 0:8]
    for j in range(1, 16):
        b = b + acc[:, j * 8:(j + 1) * 8]
    u = b[:, 0:4] + b[:, 4:8]
    t2 = u[:, 0:2] + u[:, 2:4]
    return t2[:, 0:1] + t2[:, 1:2]


def _norms_and_topk(cls_ref, pk_ref):
    x = cls_ref[...]
    xn = x * lax.rsqrt(jnp.maximum(_row_ss(x), 1e-12))
    pk = pk_ref[...]
    pkn = pk * lax.rsqrt(jnp.maximum(_row_ss(pk), 1e-12))
    sim = lax.dot_general(
        xn, pkn, (((1,), (1,)), ((), ())),
        preferred_element_type=jnp.float32,
    )
    colid = lax.broadcasted_iota(jnp.int32, (_B, _POOL), 1)
    s = sim
    iks, chosens = [], []
    for k in range(_K):
        m = jnp.max(s, axis=1, keepdims=True)
        cand = jnp.where(s >= m, colid, _POOL)
        ik = jnp.min(cand, axis=1, keepdims=True)
        chosen = colid == ik
        s = jnp.where(chosen, -jnp.inf, s)
        iks.append(ik)
        chosens.append(chosen)
    return xn, pkn, iks, chosens


def _idx_body(cls_ref, pk_ref, idxrows_ref):
    _, _, iks, _ = _norms_and_topk(cls_ref, pk_ref)
    kid16 = lax.broadcasted_iota(jnp.int32, (_B, _K, _NCH, _CH), 1)
    liota = (lax.broadcasted_iota(jnp.int32, (_B, _K, _NCH, _CH), 2) * _CH
             + lax.broadcasted_iota(jnp.int32, (_B, _K, _NCH, _CH), 3))
    pid = jnp.zeros((_B, _K, _NCH, _CH), jnp.int32)
    for k in range(_K):
        pid = jnp.where(kid16 == k, iks[k][:, :, None, None], pid)
    idxrows_ref[...] = pid * _LEN + liota


def _simout_body(cls_ref, pk_ref, sim_ref):
    xn, pkn, _, chosens = _norms_and_topk(cls_ref, pk_ref)
    kid = lax.broadcasted_iota(jnp.int32, (_B, _K, _POOL), 1)
    oh = jnp.zeros((_B, _K, _POOL), jnp.float32)
    for k in range(_K):
        oh = jnp.where(kid == k, chosens[k][:, None, :].astype(jnp.float32), oh)
    rows = lax.dot_general(
        oh.reshape(_B * _K, _POOL), pkn, (((1,), (0,)), ((), ())),
        preferred_element_type=jnp.float32, precision=lax.Precision.HIGHEST,
    )
    sim_ref[...] = rows.reshape(_B, _K, _DIM) * xn[:, None, :]


def _select_idx(cls_features, prompt_key):
    return pl.pallas_call(
        _idx_body,
        out_shape=jax.ShapeDtypeStruct((_B, _K, _NCH, _CH), jnp.int32),
    )(cls_features, prompt_key)


def _select_simout(cls_features, prompt_key):
    return pl.pallas_call(
        _simout_body,
        out_shape=jax.ShapeDtypeStruct((_B, _K, _DIM), jnp.float32),
    )(cls_features, prompt_key)


@functools.cache
def _make_gather_sc():
    @functools.partial(
        pl.kernel,
        mesh=plsc.VectorSubcoreMesh(core_axis_name="c", subcore_axis_name="s"),
        out_type=jax.ShapeDtypeStruct((_B * _K * _LEN, _DIM), jnp.float32),
        scratch_types=[
            pltpu.VMEM((_NCH, _CH), jnp.int32),
            pltpu.VMEM((_NCH, _CH, _DIM), jnp.float32),
        ] + [pltpu.SemaphoreType.DMA] * (2 * _NCH),
    )
    def _gather_sc(idxrows_hbm, table_hbm, out_hbm, rowidx_v, rows_v, *sems):
        gsems, wsems = sems[:_NCH], sems[_NCH:]
        wid = lax.axis_index("s") * 2 + lax.axis_index("c")
        pltpu.sync_copy(idxrows_hbm.at[wid // _K, wid % _K], rowidx_v)
        gathers = []
        for c in range(_NCH):
            gathers.append(pltpu.async_copy(
                table_hbm.at[rowidx_v.at[c]],
                rows_v.at[c], gsems[c]))
        writes = []
        for c in range(_NCH):
            gathers[c].wait()
            writes.append(pltpu.async_copy(
                rows_v.at[c],
                out_hbm.at[pl.ds(wid * _LEN + c * _CH, _CH)], wsems[c]))
        for wdma in writes:
            wdma.wait()

    return _gather_sc


def kernel(x_embed, cls_features, prompt, prompt_key):
    del x_embed
    idxrows = _select_idx(cls_features, prompt_key)
    table = prompt.reshape(_POOL * _LEN, _DIM)
    bp = _make_gather_sc()(idxrows, table)
    sim_out = _select_simout(cls_features, prompt_key)
    return (bp.reshape(_B, _K * _LEN, _DIM), sim_out)
